# Initial kernel scaffold; baseline (speedup 1.0000x reference)
import jax
import jax.numpy as jnp
from jax.experimental import pallas as pl


def kernel(hidden_states, embed):
    raise NotImplementedError("write your pallas kernel here")


# speedup vs baseline: 1.0142x; 1.0142x over previous
import functools

import jax
import jax.numpy as jnp
from jax import lax
from jax.experimental import pallas as pl
from jax.experimental.pallas import tpu as pltpu
from jax.experimental.pallas import tpu_sc as plsc

_N = 16384
_D = 32
_K = 8192

_NT = 512
_KT = 2048
_NB = _N // _NT
_KB = _K // _KT

_NC = 2
_NS = 16
_NW = _NC * _NS
_BPW = _N // _NW
_CHUNK = 128
_NCHUNK = _BPW // _CHUNK


_KH = _KB // 2


def _argmax_body(h_ref, et_ref, s_ref, e2_ref, idx_ref,
                 max_a, idx_a, max_b, idx_b):
    k = pl.program_id(1)
    nk = pl.num_programs(1)

    m = jnp.dot(h_ref[...], et_ref[...], preferred_element_type=jnp.float32)
    dist = -(s_ref[...] - 2.0 * m + e2_ref[...])

    mx = jnp.max(dist, axis=1, keepdims=True)
    gidx = lax.broadcasted_iota(jnp.int32, dist.shape, 1) + k * _KT
    cand = jnp.min(jnp.where(dist == mx, gidx, jnp.int32(_K)), axis=1,
                   keepdims=True)

    in_b = k >= _KH
    first = jnp.logical_or(k == 0, k == _KH)

    @pl.when(first)
    def _():
        @pl.when(in_b)
        def _():
            max_b[...] = mx
            idx_b[...] = cand

        @pl.when(jnp.logical_not(in_b))
        def _():
            max_a[...] = mx
            idx_a[...] = cand

    @pl.when(jnp.logical_not(first))
    def _():
        @pl.when(in_b)
        def _():
            better = mx > max_b[...]
            idx_b[...] = jnp.where(better, cand, idx_b[...])
            max_b[...] = jnp.where(better, mx, max_b[...])

        @pl.when(jnp.logical_not(in_b))
        def _():
            better = mx > max_a[...]
            idx_a[...] = jnp.where(better, cand, idx_a[...])
            max_a[...] = jnp.where(better, mx, max_a[...])

    @pl.when(k == nk - 1)
    def _():
        va_rounded = max_a[...].astype(jnp.bfloat16).astype(jnp.float32)
        b_wins = max_b[...] > va_rounded
        idx_ref[...] = jnp.where(b_wins, idx_b[...], idx_a[...])


def _dist_argmax(h, et, s, e2):
    return pl.pallas_call(
        _argmax_body,
        grid=(_NB, _KB),
        in_specs=[
            pl.BlockSpec((_NT, _D), lambda n, k: (n, 0)),
            pl.BlockSpec((_D, _KT), lambda n, k: (0, k)),
            pl.BlockSpec((_NT, 1), lambda n, k: (n, 0)),
            pl.BlockSpec((1, _KT), lambda n, k: (0, k)),
        ],
        out_specs=pl.BlockSpec((_NT, 1), lambda n, k: (n, 0)),
        out_shape=jax.ShapeDtypeStruct((_N, 1), jnp.int32),
        scratch_shapes=[
            pltpu.VMEM((_NT, 1), jnp.float32),
            pltpu.VMEM((_NT, 1), jnp.int32),
            pltpu.VMEM((_NT, 1), jnp.float32),
            pltpu.VMEM((_NT, 1), jnp.int32),
        ],
    )(h, et, s, e2)


_DP = 128


@functools.cache
def _sc_gather_fn():
    @functools.partial(
        pl.kernel,
        mesh=plsc.VectorSubcoreMesh(core_axis_name="c", subcore_axis_name="s"),
        out_type=jax.ShapeDtypeStruct((_N, _DP), jnp.float32),
        scratch_types=[
            pltpu.VMEM((_NCHUNK, _CHUNK), jnp.int32),
            pltpu.VMEM((_BPW, _DP), jnp.float32),
            pltpu.SemaphoreType.DMA,
        ],
    )
    def _sc_gather(idx_hbm, table_hbm, out_hbm, idx_v, rows_v, sem):
        wid = lax.axis_index("s") * _NC + lax.axis_index("c")
        pltpu.sync_copy(idx_hbm.at[pl.ds(wid * _NCHUNK, _NCHUNK)], idx_v)
        copies = [
            pltpu.async_copy(
                table_hbm.at[idx_v.at[j]],
                rows_v.at[pl.ds(j * _CHUNK, _CHUNK)],
                sem,
            )
            for j in range(_NCHUNK)
        ]
        for c in copies:
            c.wait()
        pltpu.sync_copy(rows_v, out_hbm.at[pl.ds(wid * _BPW, _BPW)])

    return _sc_gather


def kernel(hidden_states, embed):
    shape = hidden_states.shape
    h = hidden_states.reshape((-1, shape[-1]))
    et = embed.T
    scaled_states = jnp.sum(h ** 2, axis=1, keepdims=True)
    e2 = jnp.sum(et ** 2, axis=0, keepdims=True)
    idx = _dist_argmax(h.astype(jnp.bfloat16), et.astype(jnp.bfloat16),
                       scaled_states, e2)
    table_pad = jnp.pad(embed, ((0, 0), (0, _DP - _D)))
    rows = _sc_gather_fn()(idx.reshape(_N // _CHUNK, _CHUNK), table_pad)
    return rows[:, :_D].reshape(shape)


# speedup vs baseline: 1.1495x; 1.1334x over previous
import functools

import jax
import jax.numpy as jnp
from jax import lax
from jax.experimental import pallas as pl
from jax.experimental.pallas import tpu as pltpu
from jax.experimental.pallas import tpu_sc as plsc

_N = 16384
_D = 32
_K = 8192

_NT = 512
_KT = 2048
_NB = _N // _NT
_KB = _K // _KT

_NC = 2
_NS = 16
_NW = _NC * _NS
_BPW = _N // _NW
_CHUNK = 128
_NCHUNK = _BPW // _CHUNK


_KH = _KB // 2


def _argmax_body(h_ref, et_ref, s_ref, e2_ref, idx_ref,
                 max_a, idx_a, max_b, idx_b):
    k = pl.program_id(1)
    nk = pl.num_programs(1)

    m = jnp.dot(h_ref[...], et_ref[...], preferred_element_type=jnp.float32)
    dist = (2.0 * m - s_ref[...]) - e2_ref[...]

    mx = jnp.max(dist, axis=1, keepdims=True)
    gidx = lax.broadcasted_iota(jnp.int32, dist.shape, 1)
    cand = jnp.min(jnp.where(dist == mx, gidx, jnp.int32(_K)), axis=1,
                   keepdims=True) + k * _KT

    in_b = k >= _KH
    first = jnp.logical_or(k == 0, k == _KH)

    @pl.when(first)
    def _():
        @pl.when(in_b)
        def _():
            max_b[...] = mx
            idx_b[...] = cand

        @pl.when(jnp.logical_not(in_b))
        def _():
            max_a[...] = mx
            idx_a[...] = cand

    @pl.when(jnp.logical_not(first))
    def _():
        @pl.when(in_b)
        def _():
            better = mx > max_b[...]
            idx_b[...] = jnp.where(better, cand, idx_b[...])
            max_b[...] = jnp.where(better, mx, max_b[...])

        @pl.when(jnp.logical_not(in_b))
        def _():
            better = mx > max_a[...]
            idx_a[...] = jnp.where(better, cand, idx_a[...])
            max_a[...] = jnp.where(better, mx, max_a[...])

    @pl.when(k == nk - 1)
    def _():
        va_rounded = max_a[...].astype(jnp.bfloat16).astype(jnp.float32)
        b_wins = max_b[...] > va_rounded
        idx_ref[...] = jnp.where(b_wins, idx_b[...], idx_a[...])


def _dist_argmax(h, et, s, e2):
    return pl.pallas_call(
        _argmax_body,
        grid=(_NB, _KB),
        in_specs=[
            pl.BlockSpec((_NT, _D), lambda n, k: (n, 0)),
            pl.BlockSpec((_D, _KT), lambda n, k: (0, k)),
            pl.BlockSpec((_NT, 1), lambda n, k: (n, 0)),
            pl.BlockSpec((1, _KT), lambda n, k: (0, k)),
        ],
        out_specs=pl.BlockSpec((_NT, 1), lambda n, k: (n, 0)),
        out_shape=jax.ShapeDtypeStruct((_N, 1), jnp.int32),
        scratch_shapes=[
            pltpu.VMEM((_NT, 1), jnp.float32),
            pltpu.VMEM((_NT, 1), jnp.int32),
            pltpu.VMEM((_NT, 1), jnp.float32),
            pltpu.VMEM((_NT, 1), jnp.int32),
        ],
    )(h, et, s, e2)


_DP = 128


@functools.cache
def _sc_gather_fn():
    @functools.partial(
        pl.kernel,
        mesh=plsc.VectorSubcoreMesh(core_axis_name="c", subcore_axis_name="s"),
        out_type=jax.ShapeDtypeStruct((_N, _DP), jnp.float32),
        scratch_types=[
            pltpu.VMEM((_NCHUNK, _CHUNK), jnp.int32),
            pltpu.VMEM((_BPW, _DP), jnp.float32),
            pltpu.SemaphoreType.DMA,
        ],
    )
    def _sc_gather(idx_hbm, table_hbm, out_hbm, idx_v, rows_v, sem):
        wid = lax.axis_index("s") * _NC + lax.axis_index("c")
        pltpu.sync_copy(idx_hbm.at[pl.ds(wid * _NCHUNK, _NCHUNK)], idx_v)
        copies = [
            pltpu.async_copy(
                table_hbm.at[idx_v.at[j]],
                rows_v.at[pl.ds(j * _CHUNK, _CHUNK)],
                sem,
            )
            for j in range(_NCHUNK)
        ]
        for c in copies:
            c.wait()
        pltpu.sync_copy(rows_v, out_hbm.at[pl.ds(wid * _BPW, _BPW)])

    return _sc_gather


def kernel(hidden_states, embed):
    shape = hidden_states.shape
    h = hidden_states.reshape((-1, shape[-1]))
    et = embed.T
    scaled_states = jnp.sum(h ** 2, axis=1, keepdims=True)
    e2 = jnp.sum(et ** 2, axis=0, keepdims=True)
    idx = _dist_argmax(h.astype(jnp.bfloat16), et.astype(jnp.bfloat16),
                       scaled_states, e2)
    table_pad = jnp.pad(embed, ((0, 0), (0, _DP - _D)))
    rows = _sc_gather_fn()(idx.reshape(_N // _CHUNK, _CHUNK), table_pad)
    return rows[:, :_D].reshape(shape)


# speedup vs baseline: 1.2606x; 1.0967x over previous
import functools

import jax
import jax.numpy as jnp
from jax import lax
from jax.experimental import pallas as pl
from jax.experimental.pallas import tpu as pltpu
from jax.experimental.pallas import tpu_sc as plsc

_N = 16384
_D = 32
_K = 8192

_NT = 512
_KT = 2048
_NB = _N // _NT
_KB = _K // _KT

_NC = 2
_NS = 16
_NW = _NC * _NS
_BPW = _N // _NW
_CHUNK = 128
_NCHUNK = _BPW // _CHUNK


_KH = _KB // 2


def _argmax_body(h_ref, et_ref, s_ref, e2_ref, idx_ref,
                 max_a, idx_a, max_b, idx_b):
    k = pl.program_id(1)
    nk = pl.num_programs(1)

    m = jnp.dot(h_ref[...], et_ref[...], preferred_element_type=jnp.float32)
    dist = (2.0 * m - s_ref[...]) - e2_ref[...]

    _CH = 128
    lane = lax.broadcasted_iota(jnp.int32, (_NT, _CH), 1)
    acc_v = dist[:, :_CH]
    acc_i = lane
    for c in range(1, _KT // _CH):
        chunk = dist[:, c * _CH:(c + 1) * _CH]
        w = chunk > acc_v
        acc_i = jnp.where(w, lane + c * _CH, acc_i)
        acc_v = jnp.where(w, chunk, acc_v)
    mx = jnp.max(acc_v, axis=1, keepdims=True)
    cand = jnp.min(jnp.where(acc_v == mx, acc_i, jnp.int32(_K)), axis=1,
                   keepdims=True) + k * _KT

    in_b = k >= _KH
    first = jnp.logical_or(k == 0, k == _KH)

    @pl.when(first)
    def _():
        @pl.when(in_b)
        def _():
            max_b[...] = mx
            idx_b[...] = cand

        @pl.when(jnp.logical_not(in_b))
        def _():
            max_a[...] = mx
            idx_a[...] = cand

    @pl.when(jnp.logical_not(first))
    def _():
        @pl.when(in_b)
        def _():
            better = mx > max_b[...]
            idx_b[...] = jnp.where(better, cand, idx_b[...])
            max_b[...] = jnp.where(better, mx, max_b[...])

        @pl.when(jnp.logical_not(in_b))
        def _():
            better = mx > max_a[...]
            idx_a[...] = jnp.where(better, cand, idx_a[...])
            max_a[...] = jnp.where(better, mx, max_a[...])

    @pl.when(k == nk - 1)
    def _():
        va_rounded = max_a[...].astype(jnp.bfloat16).astype(jnp.float32)
        b_wins = max_b[...] > va_rounded
        idx_ref[...] = jnp.where(b_wins, idx_b[...], idx_a[...])


def _dist_argmax(h, et, s, e2):
    return pl.pallas_call(
        _argmax_body,
        grid=(_NB, _KB),
        in_specs=[
            pl.BlockSpec((_NT, _D), lambda n, k: (n, 0)),
            pl.BlockSpec((_D, _KT), lambda n, k: (0, k)),
            pl.BlockSpec((_NT, 1), lambda n, k: (n, 0)),
            pl.BlockSpec((1, _KT), lambda n, k: (0, k)),
        ],
        out_specs=pl.BlockSpec((_NT, 1), lambda n, k: (n, 0)),
        out_shape=jax.ShapeDtypeStruct((_N, 1), jnp.int32),
        scratch_shapes=[
            pltpu.VMEM((_NT, 1), jnp.float32),
            pltpu.VMEM((_NT, 1), jnp.int32),
            pltpu.VMEM((_NT, 1), jnp.float32),
            pltpu.VMEM((_NT, 1), jnp.int32),
        ],
    )(h, et, s, e2)


_DP = 128


@functools.cache
def _sc_gather_fn():
    @functools.partial(
        pl.kernel,
        mesh=plsc.VectorSubcoreMesh(core_axis_name="c", subcore_axis_name="s"),
        out_type=jax.ShapeDtypeStruct((_N, _DP), jnp.float32),
        scratch_types=[
            pltpu.VMEM((_NCHUNK, _CHUNK), jnp.int32),
            pltpu.VMEM((_BPW, _DP), jnp.float32),
            pltpu.SemaphoreType.DMA,
        ],
    )
    def _sc_gather(idx_hbm, table_hbm, out_hbm, idx_v, rows_v, sem):
        wid = lax.axis_index("s") * _NC + lax.axis_index("c")
        pltpu.sync_copy(idx_hbm.at[pl.ds(wid * _NCHUNK, _NCHUNK)], idx_v)
        copies = [
            pltpu.async_copy(
                table_hbm.at[idx_v.at[j]],
                rows_v.at[pl.ds(j * _CHUNK, _CHUNK)],
                sem,
            )
            for j in range(_NCHUNK)
        ]
        for c in copies:
            c.wait()
        pltpu.sync_copy(rows_v, out_hbm.at[pl.ds(wid * _BPW, _BPW)])

    return _sc_gather


def kernel(hidden_states, embed):
    shape = hidden_states.shape
    h = hidden_states.reshape((-1, shape[-1]))
    et = embed.T
    scaled_states = jnp.sum(h ** 2, axis=1, keepdims=True)
    e2 = jnp.sum(et ** 2, axis=0, keepdims=True)
    idx = _dist_argmax(h.astype(jnp.bfloat16), et.astype(jnp.bfloat16),
                       scaled_states, e2)
    table_pad = jnp.pad(embed, ((0, 0), (0, _DP - _D)))
    rows = _sc_gather_fn()(idx.reshape(_N // _CHUNK, _CHUNK), table_pad)
    return rows[:, :_D].reshape(shape)


# speedup vs baseline: 1.4017x; 1.1119x over previous
import functools

import jax
import jax.numpy as jnp
from jax import lax
from jax.experimental import pallas as pl
from jax.experimental.pallas import tpu as pltpu
from jax.experimental.pallas import tpu_sc as plsc

_N = 16384
_D = 32
_K = 8192

_NT = 512
_KT = 4096
_NB = _N // _NT
_KB = _K // _KT

_NC = 2
_NS = 16
_NW = _NC * _NS
_BPW = _N // _NW
_CHUNK = 128
_NCHUNK = _BPW // _CHUNK


_KH = _KB // 2


def _argmax_body(h_ref, et_ref, s_ref, e2_ref, idx_ref,
                 max_a, idx_a, max_b, idx_b):
    k = pl.program_id(1)
    nk = pl.num_programs(1)

    m = jnp.dot(h_ref[...], et_ref[...], preferred_element_type=jnp.float32)
    dist = (2.0 * m - s_ref[...]) - e2_ref[...]

    _CH = 128
    lane = lax.broadcasted_iota(jnp.int32, (_NT, _CH), 1)
    acc_v = dist[:, :_CH]
    acc_i = lane
    for c in range(1, _KT // _CH):
        chunk = dist[:, c * _CH:(c + 1) * _CH]
        w = chunk > acc_v
        acc_i = jnp.where(w, lane + c * _CH, acc_i)
        acc_v = jnp.where(w, chunk, acc_v)
    mx = jnp.max(acc_v, axis=1, keepdims=True)
    cand = jnp.min(jnp.where(acc_v == mx, acc_i, jnp.int32(_K)), axis=1,
                   keepdims=True) + k * _KT

    in_b = k >= _KH
    first = jnp.logical_or(k == 0, k == _KH)

    @pl.when(first)
    def _():
        @pl.when(in_b)
        def _():
            max_b[...] = mx
            idx_b[...] = cand

        @pl.when(jnp.logical_not(in_b))
        def _():
            max_a[...] = mx
            idx_a[...] = cand

    @pl.when(jnp.logical_not(first))
    def _():
        @pl.when(in_b)
        def _():
            better = mx > max_b[...]
            idx_b[...] = jnp.where(better, cand, idx_b[...])
            max_b[...] = jnp.where(better, mx, max_b[...])

        @pl.when(jnp.logical_not(in_b))
        def _():
            better = mx > max_a[...]
            idx_a[...] = jnp.where(better, cand, idx_a[...])
            max_a[...] = jnp.where(better, mx, max_a[...])

    @pl.when(k == nk - 1)
    def _():
        va_rounded = max_a[...].astype(jnp.bfloat16).astype(jnp.float32)
        b_wins = max_b[...] > va_rounded
        idx_ref[...] = jnp.where(b_wins, idx_b[...], idx_a[...])


def _dist_argmax(h, et, s, e2):
    return pl.pallas_call(
        _argmax_body,
        grid=(_NB, _KB),
        in_specs=[
            pl.BlockSpec((_NT, _D), lambda n, k: (n, 0)),
            pl.BlockSpec((_D, _KT), lambda n, k: (0, k)),
            pl.BlockSpec((_NT, 1), lambda n, k: (n, 0)),
            pl.BlockSpec((1, _KT), lambda n, k: (0, k)),
        ],
        out_specs=pl.BlockSpec((_NT, 1), lambda n, k: (n, 0)),
        out_shape=jax.ShapeDtypeStruct((_N, 1), jnp.int32),
        scratch_shapes=[
            pltpu.VMEM((_NT, 1), jnp.float32),
            pltpu.VMEM((_NT, 1), jnp.int32),
            pltpu.VMEM((_NT, 1), jnp.float32),
            pltpu.VMEM((_NT, 1), jnp.int32),
        ],
    )(h, et, s, e2)


_DP = 128


@functools.cache
def _sc_gather_fn():
    @functools.partial(
        pl.kernel,
        mesh=plsc.VectorSubcoreMesh(core_axis_name="c", subcore_axis_name="s"),
        out_type=jax.ShapeDtypeStruct((_N, _DP), jnp.float32),
        scratch_types=[
            pltpu.VMEM((_NCHUNK, _CHUNK), jnp.int32),
            pltpu.VMEM((_BPW, _DP), jnp.float32),
            pltpu.SemaphoreType.DMA,
        ],
    )
    def _sc_gather(idx_hbm, table_hbm, out_hbm, idx_v, rows_v, sem):
        wid = lax.axis_index("s") * _NC + lax.axis_index("c")
        pltpu.sync_copy(idx_hbm.at[pl.ds(wid * _NCHUNK, _NCHUNK)], idx_v)
        copies = [
            pltpu.async_copy(
                table_hbm.at[idx_v.at[j]],
                rows_v.at[pl.ds(j * _CHUNK, _CHUNK)],
                sem,
            )
            for j in range(_NCHUNK)
        ]
        for c in copies:
            c.wait()
        pltpu.sync_copy(rows_v, out_hbm.at[pl.ds(wid * _BPW, _BPW)])

    return _sc_gather


def kernel(hidden_states, embed):
    shape = hidden_states.shape
    h = hidden_states.reshape((-1, shape[-1]))
    et = embed.T
    scaled_states = jnp.sum(h ** 2, axis=1, keepdims=True)
    e2 = jnp.sum(et ** 2, axis=0, keepdims=True)
    idx = _dist_argmax(h.astype(jnp.bfloat16), et.astype(jnp.bfloat16),
                       scaled_states, e2)
    table_pad = jnp.pad(embed, ((0, 0), (0, _DP - _D)))
    rows = _sc_gather_fn()(idx.reshape(_N // _CHUNK, _CHUNK), table_pad)
    return rows[:, :_D].reshape(shape)


# speedup vs baseline: 1.4957x; 1.0671x over previous
import functools

import jax
import jax.numpy as jnp
from jax import lax
from jax.experimental import pallas as pl
from jax.experimental.pallas import tpu as pltpu
from jax.experimental.pallas import tpu_sc as plsc

_N = 16384
_D = 32
_K = 8192

_NT = 1024
_KT = 4096
_NB = _N // _NT
_KB = _K // _KT

_NC = 2
_NS = 16
_NW = _NC * _NS
_BPW = _N // _NW
_CHUNK = 128
_NCHUNK = _BPW // _CHUNK


_KH = _KB // 2


def _argmax_body(h_ref, et_ref, s_ref, e2_ref, idx_ref,
                 max_a, idx_a, max_b, idx_b):
    k = pl.program_id(1)
    nk = pl.num_programs(1)

    m = jnp.dot(h_ref[...], et_ref[...], preferred_element_type=jnp.float32)
    dist = (2.0 * m - s_ref[...]) - e2_ref[...]

    _CH = 128
    lane = lax.broadcasted_iota(jnp.int32, (_NT, _CH), 1)
    acc_v = dist[:, :_CH]
    acc_i = lane
    for c in range(1, _KT // _CH):
        chunk = dist[:, c * _CH:(c + 1) * _CH]
        w = chunk > acc_v
        acc_i = jnp.where(w, lane + c * _CH, acc_i)
        acc_v = jnp.where(w, chunk, acc_v)
    mx = jnp.max(acc_v, axis=1, keepdims=True)
    cand = jnp.min(jnp.where(acc_v == mx, acc_i, jnp.int32(_K)), axis=1,
                   keepdims=True) + k * _KT

    in_b = k >= _KH
    first = jnp.logical_or(k == 0, k == _KH)

    @pl.when(first)
    def _():
        @pl.when(in_b)
        def _():
            max_b[...] = mx
            idx_b[...] = cand

        @pl.when(jnp.logical_not(in_b))
        def _():
            max_a[...] = mx
            idx_a[...] = cand

    @pl.when(jnp.logical_not(first))
    def _():
        @pl.when(in_b)
        def _():
            better = mx > max_b[...]
            idx_b[...] = jnp.where(better, cand, idx_b[...])
            max_b[...] = jnp.where(better, mx, max_b[...])

        @pl.when(jnp.logical_not(in_b))
        def _():
            better = mx > max_a[...]
            idx_a[...] = jnp.where(better, cand, idx_a[...])
            max_a[...] = jnp.where(better, mx, max_a[...])

    @pl.when(k == nk - 1)
    def _():
        va_rounded = max_a[...].astype(jnp.bfloat16).astype(jnp.float32)
        b_wins = max_b[...] > va_rounded
        idx_ref[...] = jnp.where(b_wins, idx_b[...], idx_a[...])


def _dist_argmax(h, et, s, e2):
    return pl.pallas_call(
        _argmax_body,
        grid=(_NB, _KB),
        in_specs=[
            pl.BlockSpec((_NT, _D), lambda n, k: (n, 0)),
            pl.BlockSpec((_D, _KT), lambda n, k: (0, k)),
            pl.BlockSpec((_NT, 1), lambda n, k: (n, 0)),
            pl.BlockSpec((1, _KT), lambda n, k: (0, k)),
        ],
        out_specs=pl.BlockSpec((_NT, 1), lambda n, k: (n, 0)),
        out_shape=jax.ShapeDtypeStruct((_N, 1), jnp.int32),
        scratch_shapes=[
            pltpu.VMEM((_NT, 1), jnp.float32),
            pltpu.VMEM((_NT, 1), jnp.int32),
            pltpu.VMEM((_NT, 1), jnp.float32),
            pltpu.VMEM((_NT, 1), jnp.int32),
        ],
    )(h, et, s, e2)


_DP = 128


@functools.cache
def _sc_gather_fn():
    @functools.partial(
        pl.kernel,
        mesh=plsc.VectorSubcoreMesh(core_axis_name="c", subcore_axis_name="s"),
        out_type=jax.ShapeDtypeStruct((_N, _DP), jnp.float32),
        scratch_types=[
            pltpu.VMEM((_NCHUNK, _CHUNK), jnp.int32),
            pltpu.VMEM((_BPW, _DP), jnp.float32),
            pltpu.SemaphoreType.DMA,
        ],
    )
    def _sc_gather(idx_hbm, table_hbm, out_hbm, idx_v, rows_v, sem):
        wid = lax.axis_index("s") * _NC + lax.axis_index("c")
        pltpu.sync_copy(idx_hbm.at[pl.ds(wid * _NCHUNK, _NCHUNK)], idx_v)
        copies = [
            pltpu.async_copy(
                table_hbm.at[idx_v.at[j]],
                rows_v.at[pl.ds(j * _CHUNK, _CHUNK)],
                sem,
            )
            for j in range(_NCHUNK)
        ]
        for c in copies:
            c.wait()
        pltpu.sync_copy(rows_v, out_hbm.at[pl.ds(wid * _BPW, _BPW)])

    return _sc_gather


def kernel(hidden_states, embed):
    shape = hidden_states.shape
    h = hidden_states.reshape((-1, shape[-1]))
    et = embed.T
    scaled_states = jnp.sum(h ** 2, axis=1, keepdims=True)
    e2 = jnp.sum(et ** 2, axis=0, keepdims=True)
    idx = _dist_argmax(h.astype(jnp.bfloat16), et.astype(jnp.bfloat16),
                       scaled_states, e2)
    table_pad = jnp.pad(embed, ((0, 0), (0, _DP - _D)))
    rows = _sc_gather_fn()(idx.reshape(_N // _CHUNK, _CHUNK), table_pad)
    return rows[:, :_D].reshape(shape)


# speedup vs baseline: 1.5438x; 1.0322x over previous
import functools

import jax
import jax.numpy as jnp
from jax import lax
from jax.experimental import pallas as pl
from jax.experimental.pallas import tpu as pltpu
from jax.experimental.pallas import tpu_sc as plsc

_N = 16384
_D = 32
_K = 8192

_NT = 2048
_KT = 4096
_NB = _N // _NT
_KB = _K // _KT

_NC = 2
_NS = 16
_NW = _NC * _NS
_BPW = _N // _NW
_CHUNK = 128
_NCHUNK = _BPW // _CHUNK


_KH = _KB // 2


def _argmax_body(h_ref, et_ref, s_ref, e2_ref, idx_ref,
                 max_a, idx_a, max_b, idx_b):
    k = pl.program_id(1)
    nk = pl.num_programs(1)

    m = jnp.dot(h_ref[...], et_ref[...], preferred_element_type=jnp.float32)
    dist = (2.0 * m - s_ref[...]) - e2_ref[...]

    _CH = 128
    lane = lax.broadcasted_iota(jnp.int32, (_NT, _CH), 1)
    acc_v = dist[:, :_CH]
    acc_i = lane
    for c in range(1, _KT // _CH):
        chunk = dist[:, c * _CH:(c + 1) * _CH]
        w = chunk > acc_v
        acc_i = jnp.where(w, lane + c * _CH, acc_i)
        acc_v = jnp.where(w, chunk, acc_v)
    mx = jnp.max(acc_v, axis=1, keepdims=True)
    cand = jnp.min(jnp.where(acc_v == mx, acc_i, jnp.int32(_K)), axis=1,
                   keepdims=True) + k * _KT

    in_b = k >= _KH
    first = jnp.logical_or(k == 0, k == _KH)

    @pl.when(first)
    def _():
        @pl.when(in_b)
        def _():
            max_b[...] = mx
            idx_b[...] = cand

        @pl.when(jnp.logical_not(in_b))
        def _():
            max_a[...] = mx
            idx_a[...] = cand

    @pl.when(jnp.logical_not(first))
    def _():
        @pl.when(in_b)
        def _():
            better = mx > max_b[...]
            idx_b[...] = jnp.where(better, cand, idx_b[...])
            max_b[...] = jnp.where(better, mx, max_b[...])

        @pl.when(jnp.logical_not(in_b))
        def _():
            better = mx > max_a[...]
            idx_a[...] = jnp.where(better, cand, idx_a[...])
            max_a[...] = jnp.where(better, mx, max_a[...])

    @pl.when(k == nk - 1)
    def _():
        va_rounded = max_a[...].astype(jnp.bfloat16).astype(jnp.float32)
        b_wins = max_b[...] > va_rounded
        idx_ref[...] = jnp.where(b_wins, idx_b[...], idx_a[...])


def _dist_argmax(h, et, s, e2):
    return pl.pallas_call(
        _argmax_body,
        grid=(_NB, _KB),
        in_specs=[
            pl.BlockSpec((_NT, _D), lambda n, k: (n, 0)),
            pl.BlockSpec((_D, _KT), lambda n, k: (0, k)),
            pl.BlockSpec((_NT, 1), lambda n, k: (n, 0)),
            pl.BlockSpec((1, _KT), lambda n, k: (0, k)),
        ],
        out_specs=pl.BlockSpec((_NT, 1), lambda n, k: (n, 0)),
        out_shape=jax.ShapeDtypeStruct((_N, 1), jnp.int32),
        scratch_shapes=[
            pltpu.VMEM((_NT, 1), jnp.float32),
            pltpu.VMEM((_NT, 1), jnp.int32),
            pltpu.VMEM((_NT, 1), jnp.float32),
            pltpu.VMEM((_NT, 1), jnp.int32),
        ],
    )(h, et, s, e2)


_DP = 128


@functools.cache
def _sc_gather_fn():
    @functools.partial(
        pl.kernel,
        mesh=plsc.VectorSubcoreMesh(core_axis_name="c", subcore_axis_name="s"),
        out_type=jax.ShapeDtypeStruct((_N, _DP), jnp.float32),
        scratch_types=[
            pltpu.VMEM((_NCHUNK, _CHUNK), jnp.int32),
            pltpu.VMEM((_BPW, _DP), jnp.float32),
            pltpu.SemaphoreType.DMA,
        ],
    )
    def _sc_gather(idx_hbm, table_hbm, out_hbm, idx_v, rows_v, sem):
        wid = lax.axis_index("s") * _NC + lax.axis_index("c")
        pltpu.sync_copy(idx_hbm.at[pl.ds(wid * _NCHUNK, _NCHUNK)], idx_v)
        copies = [
            pltpu.async_copy(
                table_hbm.at[idx_v.at[j]],
                rows_v.at[pl.ds(j * _CHUNK, _CHUNK)],
                sem,
            )
            for j in range(_NCHUNK)
        ]
        for c in copies:
            c.wait()
        pltpu.sync_copy(rows_v, out_hbm.at[pl.ds(wid * _BPW, _BPW)])

    return _sc_gather


def kernel(hidden_states, embed):
    shape = hidden_states.shape
    h = hidden_states.reshape((-1, shape[-1]))
    et = embed.T
    scaled_states = jnp.sum(h ** 2, axis=1, keepdims=True)
    e2 = jnp.sum(et ** 2, axis=0, keepdims=True)
    idx = _dist_argmax(h.astype(jnp.bfloat16), et.astype(jnp.bfloat16),
                       scaled_states, e2)
    table_pad = jnp.pad(embed, ((0, 0), (0, _DP - _D)))
    rows = _sc_gather_fn()(idx.reshape(_N // _CHUNK, _CHUNK), table_pad)
    return rows[:, :_D].reshape(shape)
